# Initial kernel scaffold; baseline (speedup 1.0000x reference)
import functools

import numpy as np
import jax
import jax.numpy as jnp
from jax import lax
from jax.experimental import pallas as pl
from jax.experimental.pallas import tpu as pltpu
from jax.experimental.pallas import tpu_sc as plsc

_V = 100000
_D = 64
_B = 4096
_S = 200
_N = _B * _S

_NC, _NS, _L = 2, 16, 16
_NW = _NC * _NS
_NPW = _N // _NW
_C = _S
_NCHUNK = _NPW // _C
_SCALE = 8.0


def _pos_encoding_np():
    pos = np.arange(_S)[:, np.newaxis]
    i = np.arange(_D)[np.newaxis, :]
    angle_rates = 1 / np.power(10000, 2 * (i // 2) / np.float32(_D))
    ang = pos * angle_rates
    ang[:, 0::2] = np.sin(ang[:, 0::2])
    ang[:, 1::2] = np.cos(ang[:, 1::2])
    return np.asarray(ang, dtype=np.float32)


_PE = jnp.asarray(_pos_encoding_np())


def _make_sc_kernel():
    mesh = plsc.VectorSubcoreMesh(core_axis_name="c", subcore_axis_name="s")

    @functools.partial(
        pl.kernel,
        mesh=mesh,
        out_type=jax.ShapeDtypeStruct((_N, _D), jnp.float32),
        scratch_types=[
            pltpu.VMEM((_C,), jnp.int32),
            pltpu.VMEM((_C, _D), jnp.float32),
            pltpu.VMEM((_S, _D), jnp.float32),
            pltpu.SemaphoreType.DMA,
        ],
    )
    def k(x_hbm, table_hbm, pe_hbm, out_hbm, idx_v, rows_v, pe_v, sem):
        wid = lax.axis_index("s") * _NC + lax.axis_index("c")
        base = wid * _NPW
        pltpu.sync_copy(pe_hbm, pe_v)

        def chunk(c, carry):
            off = base + c * _C
            pltpu.sync_copy(x_hbm.at[pl.ds(off, _C)], idx_v)
            h = _C // 2
            cp0 = pltpu.async_copy(
                table_hbm.at[idx_v.at[pl.ds(0, h)]], rows_v.at[pl.ds(0, h)], sem)
            cp1 = pltpu.async_copy(
                table_hbm.at[idx_v.at[pl.ds(h, h)]], rows_v.at[pl.ds(h, h)], sem)
            cp0.wait()
            cp1.wait()

            def row(r, carry2):
                for j in range(_D // _L):
                    sl = pl.ds(j * _L, _L)
                    rows_v[r, sl] = rows_v[r, sl] * _SCALE + pe_v[r, sl]
                return carry2

            lax.fori_loop(0, _C, row, 0)
            pltpu.sync_copy(rows_v, out_hbm.at[pl.ds(off, _C)])
            return carry

        lax.fori_loop(0, _NCHUNK, chunk, 0)

    return k


_sc_kernel = _make_sc_kernel()


def kernel(x, table):
    out = _sc_kernel(x.reshape(_N), table, _PE)
    return out.reshape(_B, _S, _D)


# speedup vs baseline: 3.0422x; 3.0422x over previous
import functools

import numpy as np
import jax
import jax.numpy as jnp
from jax import lax
from jax.experimental import pallas as pl
from jax.experimental.pallas import tpu as pltpu
from jax.experimental.pallas import tpu_sc as plsc

_V = 100000
_D = 64
_B = 4096
_S = 200
_N = _B * _S

_NC, _NS, _L = 2, 16, 16
_NW = _NC * _NS
_NPW = _N // _NW
_C = _S
_NCHUNK = _NPW // _C
_SCALE = 8.0


def _pos_encoding_np():
    pos = np.arange(_S)[:, np.newaxis]
    i = np.arange(_D)[np.newaxis, :]
    angle_rates = 1 / np.power(10000, 2 * (i // 2) / np.float32(_D))
    ang = pos * angle_rates
    ang[:, 0::2] = np.sin(ang[:, 0::2])
    ang[:, 1::2] = np.cos(ang[:, 1::2])
    return np.asarray(ang, dtype=np.float32)


_PE = _pos_encoding_np()


def _make_sc_kernel():
    mesh = plsc.VectorSubcoreMesh(core_axis_name="c", subcore_axis_name="s")

    @functools.partial(
        pl.kernel,
        mesh=mesh,
        out_type=jax.ShapeDtypeStruct((_N, _D), jnp.float32),
        compiler_params=pltpu.CompilerParams(use_tc_tiling_on_sc=False),
        scratch_types=[
            pltpu.VMEM((_C,), jnp.int32),
            pltpu.VMEM((_C, _D), jnp.float32),
            pltpu.VMEM((_S, _D), jnp.float32),
            pltpu.SemaphoreType.DMA,
        ],
    )
    def k(x_hbm, table_hbm, pe_hbm, out_hbm, idx_v, rows_v, pe_v, sem):
        wid = lax.axis_index("s") * _NC + lax.axis_index("c")
        base = wid * _NPW
        pltpu.sync_copy(pe_hbm, pe_v)

        def chunk(c, carry):
            off = base + c * _C
            pltpu.sync_copy(x_hbm.at[pl.ds(off, _C)], idx_v)
            h = 104
            cp0 = pltpu.async_copy(
                table_hbm.at[idx_v.at[pl.ds(0, h)]], rows_v.at[pl.ds(0, h)], sem)
            cp1 = pltpu.async_copy(
                table_hbm.at[idx_v.at[pl.ds(h, _C - h)]],
                rows_v.at[pl.ds(h, _C - h)], sem)
            cp0.wait()
            cp1.wait()

            def row(r, carry2):
                for j in range(_D // _L):
                    sl = pl.ds(j * _L, _L)
                    rows_v[r, sl] = rows_v[r, sl] * _SCALE + pe_v[r, sl]
                return carry2

            lax.fori_loop(0, _C, row, 0)
            pltpu.sync_copy(rows_v, out_hbm.at[pl.ds(off, _C)])
            return carry

        lax.fori_loop(0, _NCHUNK, chunk, 0)

    return k


_SC_KERNEL_CACHE = []


def _get_sc_kernel():
    if not _SC_KERNEL_CACHE:
        _SC_KERNEL_CACHE.append(_make_sc_kernel())
    return _SC_KERNEL_CACHE[0]


def kernel(x, table):
    out = _get_sc_kernel()(x.reshape(_N), table, jnp.asarray(_PE))
    return out.reshape(_B, _S, _D)
